# baseline (speedup 1.0000x reference)
import jax
import jax.numpy as jnp
from jax import lax
from jax.experimental import pallas as pl
from jax.experimental.pallas import tpu as pltpu
from jax.experimental.pallas import tpu_sc as plsc

B = 16384
F = 26
FP = 32
D = 16
NC = 2
NS = 16
NW = NC * NS
RPW = B // NW
C = 64
NCHUNK = RPW // C
EC = C * F
PC = C * FP
GSZ = 128


def _fm_body(idx26_hbm, idx32_hbm, val32_hbm, embed_hbm, w_hbm, b16_hbm,
             out_hbm, idx26_v, idx32_v, val_v, rows_v, w_v, out_v, comb_v,
             b_v, sem):
    wid = lax.axis_index("s") * NC + lax.axis_index("c")
    row0 = wid * RPW
    pltpu.sync_copy(b16_hbm, b_v)
    breg = b_v[...]

    def chunk_body(c, carry):
        e_base = pl.multiple_of((row0 + c * C) * F, EC)
        p_base = pl.multiple_of((row0 + c * C) * FP, PC)
        pltpu.sync_copy(idx26_hbm.at[pl.ds(e_base, EC)], idx26_v)
        pltpu.sync_copy(idx32_hbm.at[pl.ds(p_base, PC)], idx32_v)
        pltpu.sync_copy(val32_hbm.at[pl.ds(p_base, PC)], val_v)
        copies = []
        for i in range(EC // GSZ):
            copies.append(pltpu.async_copy(
                embed_hbm.at[idx26_v.at[pl.ds(i * GSZ, GSZ)]],
                rows_v.at[pl.ds(i * GSZ, GSZ), :], sem))
        for i in range(PC // GSZ):
            copies.append(pltpu.async_copy(
                w_hbm.at[idx32_v.at[pl.ds(i * GSZ, GSZ)]],
                w_v.at[pl.ds(i * GSZ, GSZ)], sem))
        for cp in copies:
            cp.wait()

        def row_body(r, rcarry):
            fb26 = r * F
            fb32 = r * FP
            v0 = val_v[pl.ds(fb32, D)]
            v1 = val_v[pl.ds(fb32 + D, D)]
            w0 = w_v[pl.ds(fb32, D)]
            w1 = w_v[pl.ds(fb32 + D, D)]
            s = jnp.zeros((D,), jnp.float32)
            q = jnp.zeros((D,), jnp.float32)
            for j in range(F):
                e = rows_v[fb26 + j]
                vj = v0[j] if j < D else v1[j - D]
                t = e * vj
                s = s + t
                q = q + t * t
            comb_v[pl.ds(r * D, D)] = 0.5 * (s * s - q) + v0 * w0 + v1 * w1 + breg
            return rcarry

        lax.fori_loop(0, C, row_body, 0)

        flat_iota = lax.iota(jnp.int32, D) * D
        for g in range(C // D):
            base = flat_iota + g * D * D
            acc = plsc.load_gather(comb_v, [base])
            for l in range(1, D):
                acc = acc + plsc.load_gather(comb_v, [base + l])
            out_v[pl.ds(c * C + g * D, D)] = acc
        return carry

    lax.fori_loop(0, NCHUNK, chunk_body, 0)
    pltpu.sync_copy(out_v, out_hbm.at[pl.ds(pl.multiple_of(row0, RPW), RPW)])


@jax.jit
def _fm(idx26, idx32, val32, feature_embed, linear_w, b16):
    fm = pl.kernel(
        _fm_body,
        out_type=jax.ShapeDtypeStruct((B,), jnp.float32),
        mesh=plsc.VectorSubcoreMesh(core_axis_name="c", subcore_axis_name="s"),
        compiler_params=pltpu.CompilerParams(
            needs_layout_passes=False, use_tc_tiling_on_sc=False),
        scratch_types=[
            pltpu.VMEM((EC,), jnp.int32),
            pltpu.VMEM((PC,), jnp.int32),
            pltpu.VMEM((PC,), jnp.float32),
            pltpu.VMEM((EC, D), jnp.float32),
            pltpu.VMEM((PC,), jnp.float32),
            pltpu.VMEM((RPW,), jnp.float32),
            pltpu.VMEM((C * D,), jnp.float32),
            pltpu.VMEM((D,), jnp.float32),
            pltpu.SemaphoreType.DMA,
        ],
    )
    return fm(idx26, idx32, val32, feature_embed, linear_w, b16)


def kernel(feat_idx, feat_val, feature_embed, linear_w, linear_b):
    idx26 = feat_idx.reshape(-1).astype(jnp.int32)
    idx32 = jnp.pad(feat_idx.astype(jnp.int32), ((0, 0), (0, FP - F))).reshape(-1)
    val32 = jnp.pad(feat_val, ((0, 0), (0, FP - F))).reshape(-1)
    b16 = jnp.full((D,), linear_b / D, dtype=jnp.float32)
    return _fm(idx26, idx32, val32, feature_embed, linear_w, b16)
